# baseline (speedup 1.0000x reference)
import functools

import numpy as np
import jax
import jax.numpy as jnp
from jax import lax
from jax.experimental import pallas as pl
from jax.experimental.pallas import tpu as pltpu
from jax.experimental.pallas import tpu_sc as plsc

_B, _C, _H, _W = 4, 96, 224, 224
_PAD, _S, _K = 1, 4, 6
_NH, _NW = _H // _S, _W // _S
_P = _B * _NH * _NW
_KK = _K * _K
_HP = 232
_WPAD = 256
_PPB = _NH * _NW

_PBLK = 128
_NB = _P // _PBLK
_NCG = 8
_NGRP = _C // _NCG
_NWORK = 32
_NTASK = _NB * _NGRP
_NSLOT = -(-_NTASK // _NWORK)
_NROW = 24
_KP = 9
_NPIECE = _KK // _KP
_CROSS = (24, 73)


def _build_patch_table():
    p = np.arange(_P)
    hp = (p % _PPB) // _NW
    wp = p % _NW
    pb = p // _PBLK
    b = p // _PPB
    p0 = pb * _PBLK
    b0 = p0 // _PPB
    hp_min = (p0 % _PPB) // _NW
    base = 8 * (hp_min // 2)
    is_cross = np.isin(pb, _CROSS)
    second = is_cross & (b > b0)
    row = np.where(is_cross,
                   np.where(second, 4 * hp, 4 * hp - 216),
                   4 * hp - base)
    return ((row << 8) | (4 * wp)).astype(np.int32)


_PTBL = _build_patch_table()

_mesh = plsc.VectorSubcoreMesh(core_axis_name="c", subcore_axis_name="s")


@functools.partial(
    pl.kernel,
    out_type=jax.ShapeDtypeStruct((_KK, _C, _P), jnp.float32),
    mesh=_mesh,
    compiler_params=pltpu.CompilerParams(needs_layout_passes=False),
    scratch_types=[
        pltpu.VMEM((2, _PBLK), jnp.int32),
        pltpu.VMEM((2, _NCG, _NROW, _WPAD), jnp.float32),
        pltpu.VMEM((2, _KP, _NCG, _PBLK), jnp.float32),
        pltpu.SemaphoreType.DMA,
        pltpu.SemaphoreType.DMA,
        pltpu.SemaphoreType.DMA,
    ],
)
def _extract_patches(xpad_hbm, tbl_hbm, out_hbm,
                     tbl_v, in_v, out_v, sem_tbl, sem_in, sem_out):
    wid = lax.axis_index("s") * 2 + lax.axis_index("c")
    cvecs = [jnp.full((16,), c, jnp.int32) for c in range(_NCG)]

    def scalars(t):
        g = lax.rem(t, _NGRP)
        pb = lax.div(t, _NGRP)
        p0 = pb * _PBLK
        b0 = lax.div(p0, _PPB)
        hp_min = lax.div(lax.rem(p0, _PPB), _NW)
        base = 8 * lax.div(hp_min, 2)
        c0 = g * _NCG
        crossing = jnp.logical_or(pb == _CROSS[0], pb == _CROSS[1])
        return p0, b0, base, c0, crossing

    def in_copies(t, par):
        p0, b0, base, c0, crossing = scalars(t)
        tc = pltpu.make_async_copy(
            tbl_hbm.at[pl.ds(p0, _PBLK)], tbl_v.at[par], sem_tbl)
        xc = pltpu.make_async_copy(
            xpad_hbm.at[b0, pl.ds(c0, _NCG), pl.ds(216, 16), :],
            in_v.at[par, :, pl.ds(0, 16), :], sem_in)
        nc = pltpu.make_async_copy(
            xpad_hbm.at[b0, pl.ds(c0, _NCG), pl.ds(base, _NROW), :],
            in_v.at[par, :, pl.ds(0, _NROW), :], sem_in)
        return tc, xc, nc, crossing

    def issue_in(t, par):
        tc, xc, nc, crossing = in_copies(t, par)
        tc.start()
        pl.when(crossing)(lambda: xc.start())
        pl.when(jnp.logical_not(crossing))(lambda: nc.start())

    def wait_in(t, par):
        tc, xc, nc, crossing = in_copies(t, par)
        tc.wait()
        pl.when(crossing)(lambda: xc.wait())
        pl.when(jnp.logical_not(crossing))(lambda: nc.wait())

    issue_in(wid, 0)

    def slot_body(slot, carry):
        t = slot * _NWORK + wid
        par = lax.rem(slot, 2)

        @pl.when(t < _NTASK)
        def _run():
            p0, b0, base, c0, crossing = scalars(t)
            wait_in(t, par)

            @pl.when(t + _NWORK < _NTASK)
            def _prefetch():
                issue_in(t + _NWORK, 1 - par)

            tv = [tbl_v[par, pl.ds(16 * v, 16)] for v in range(8)]
            tvr = [lax.shift_right_logical(x, 8) for x in tv]
            tvc = [lax.bitwise_and(x, 255) for x in tv]
            in_cur = in_v.at[par]

            def gather_piece(kp, bsel, v_lo, v_hi):
                @plsc.parallel_loop(0, _KP, unroll=3)
                def body(kl):
                    k = kp * _KP + kl
                    i = lax.div(k, _K)
                    j = lax.rem(k, _K)
                    ir = [tvr[v] + i for v in range(v_lo, v_hi)]
                    ico = [tvc[v] + j for v in range(v_lo, v_hi)]
                    for c in range(_NCG):
                        for vv, v in enumerate(range(v_lo, v_hi)):
                            out_v[bsel, kl, c, pl.ds(16 * v, 16)] = (
                                plsc.load_gather(
                                    in_cur, [cvecs[c], ir[vv], ico[vv]]))

            def piece_body(kp, cw):
                bsel = lax.rem(kp, 2)
                out_dma = pltpu.make_async_copy(
                    out_v.at[bsel],
                    out_hbm.at[pl.ds(kp * _KP, _KP), pl.ds(c0, _NCG),
                               pl.ds(p0, _PBLK)], sem_out)

                @pl.when(jnp.logical_or(slot > 0, kp >= 2))
                def _drain():
                    pltpu.make_async_copy(
                        out_v.at[bsel],
                        out_hbm.at[pl.ds(0, _KP), pl.ds(c0, _NCG),
                                   pl.ds(p0, _PBLK)], sem_out).wait()

                @pl.when(jnp.logical_not(crossing))
                def _normal():
                    gather_piece(kp, bsel, 0, 8)

                @pl.when(crossing)
                def _cross():
                    @pl.when(kp > 0)
                    def _restage1():
                        pltpu.sync_copy(
                            xpad_hbm.at[b0, pl.ds(c0, _NCG),
                                        pl.ds(216, 16), :],
                            in_v.at[par, :, pl.ds(0, 16), :])

                    gather_piece(kp, bsel, 0, 4)
                    pltpu.sync_copy(
                        xpad_hbm.at[b0 + 1, pl.ds(c0, _NCG),
                                    pl.ds(0, 16), :],
                        in_v.at[par, :, pl.ds(0, 16), :])
                    gather_piece(kp, bsel, 4, 8)

                out_dma.start()
                return cw

            lax.fori_loop(0, _NPIECE, piece_body, 0)

        return carry

    lax.fori_loop(0, _NSLOT, slot_body, 0)

    for _ in range(2):
        pltpu.make_async_copy(
            out_v.at[0],
            out_hbm.at[pl.ds(0, _KP), pl.ds(0, _NCG), pl.ds(0, _PBLK)],
            sem_out).wait()


def kernel(out_lr):
    xpad = jnp.pad(out_lr, ((0, 0), (0, 0), (_PAD, _HP - _H - _PAD),
                            (_PAD, _WPAD - _W - _PAD)))
    out_t = _extract_patches(xpad, jnp.asarray(_PTBL))
    patches = jnp.transpose(out_t.reshape(_K, _K, _C, _P), (3, 2, 0, 1))
    b_idx = jnp.repeat(jnp.arange(_B, dtype=jnp.int32), _NH * _NW)
    h_idx = jnp.tile(jnp.repeat(jnp.arange(_NH, dtype=jnp.int32), _NW), _B)
    w_idx = jnp.tile(jnp.arange(_NW, dtype=jnp.int32), _B * _NH)
    return (patches, (b_idx, h_idx, w_idx))


# speedup vs baseline: 2.0168x; 2.0168x over previous
import functools

import numpy as np
import jax
import jax.numpy as jnp
from jax import lax
from jax.experimental import pallas as pl
from jax.experimental.pallas import tpu as pltpu
from jax.experimental.pallas import tpu_sc as plsc

_B, _C, _H, _W = 4, 96, 224, 224
_PAD, _S, _K = 1, 4, 6
_NH, _NW = _H // _S, _W // _S
_P = _B * _NH * _NW
_KK = _K * _K
_HP = 232
_WPAD = 256
_PPB = _NH * _NW

_PBLK = 128
_NB = _P // _PBLK
_NCG = 8
_NGRP = _C // _NCG
_NWORK = 32
_NTASK = _NB * _NGRP
_NSLOT = -(-_NTASK // _NWORK)
_NROW = 24
_KP = 9
_NPIECE = _KK // _KP
_CROSS = (24, 73)


def _build_patch_table():
    p = np.arange(_P)
    hp = (p % _PPB) // _NW
    wp = p % _NW
    pb = p // _PBLK
    b = p // _PPB
    p0 = pb * _PBLK
    b0 = p0 // _PPB
    hp_min = (p0 % _PPB) // _NW
    base = 8 * (hp_min // 2)
    is_cross = np.isin(pb, _CROSS)
    second = is_cross & (b > b0)
    row = np.where(is_cross,
                   np.where(second, 4 * hp, 4 * hp - 216),
                   4 * hp - base)
    return ((row << 8) | (4 * wp)).astype(np.int32)


_PTBL = _build_patch_table()

_mesh = plsc.VectorSubcoreMesh(core_axis_name="c", subcore_axis_name="s")


@functools.partial(
    pl.kernel,
    out_type=jax.ShapeDtypeStruct((_KK, _C, _P), jnp.float32),
    mesh=_mesh,
    compiler_params=pltpu.CompilerParams(needs_layout_passes=False),
    scratch_types=[
        pltpu.VMEM((2, _PBLK), jnp.int32),
        pltpu.VMEM((2, _NCG, _NROW, _WPAD), jnp.float32),
        pltpu.VMEM((2, _KP, _NCG, _PBLK), jnp.float32),
        pltpu.SemaphoreType.DMA,
        pltpu.SemaphoreType.DMA,
        pltpu.SemaphoreType.DMA,
    ],
)
def _extract_patches(xpad_hbm, tbl_hbm, out_hbm,
                     tbl_v, in_v, out_v, sem_tbl, sem_in, sem_out):
    wid = lax.axis_index("s") * 2 + lax.axis_index("c")
    cvecs = [jnp.full((16,), c, jnp.int32) for c in range(_NCG)]

    def scalars(t):
        g = lax.rem(t, _NGRP)
        pb = lax.div(t, _NGRP)
        p0 = pb * _PBLK
        b0 = lax.div(p0, _PPB)
        hp_min = lax.div(lax.rem(p0, _PPB), _NW)
        base = 8 * lax.div(hp_min, 2)
        c0 = g * _NCG
        crossing = jnp.logical_or(pb == _CROSS[0], pb == _CROSS[1])
        return p0, b0, base, c0, crossing

    def in_copies(t, par):
        p0, b0, base, c0, crossing = scalars(t)
        tc = pltpu.make_async_copy(
            tbl_hbm.at[pl.ds(p0, _PBLK)], tbl_v.at[par], sem_tbl)
        xc = pltpu.make_async_copy(
            xpad_hbm.at[b0, pl.ds(c0, _NCG), pl.ds(216, 16), :],
            in_v.at[par, :, pl.ds(0, 16), :], sem_in)
        nc = pltpu.make_async_copy(
            xpad_hbm.at[b0, pl.ds(c0, _NCG), pl.ds(base, _NROW), :],
            in_v.at[par, :, pl.ds(0, _NROW), :], sem_in)
        return tc, xc, nc, crossing

    def issue_in(t, par):
        tc, xc, nc, crossing = in_copies(t, par)
        tc.start()
        pl.when(crossing)(lambda: xc.start())
        pl.when(jnp.logical_not(crossing))(lambda: nc.start())

    def wait_in(t, par):
        tc, xc, nc, crossing = in_copies(t, par)
        tc.wait()
        pl.when(crossing)(lambda: xc.wait())
        pl.when(jnp.logical_not(crossing))(lambda: nc.wait())

    issue_in(wid, 0)

    def slot_body(slot, carry):
        t = slot * _NWORK + wid
        par = lax.rem(slot, 2)

        @pl.when(t < _NTASK)
        def _run():
            p0, b0, base, c0, crossing = scalars(t)
            wait_in(t, par)

            @pl.when(t + _NWORK < _NTASK)
            def _prefetch():
                issue_in(t + _NWORK, 1 - par)

            tv = [tbl_v[par, pl.ds(16 * v, 16)] for v in range(8)]
            tvr = [lax.shift_right_logical(x, 8) for x in tv]
            tvc = [lax.bitwise_and(x, 255) for x in tv]
            in_cur = in_v.at[par]

            def gather_piece(kp, bsel, v_lo, v_hi):
                @plsc.parallel_loop(0, _KP, unroll=1)
                def body(kl):
                    k = kp * _KP + kl
                    i = lax.div(k, _K)
                    j = lax.rem(k, _K)
                    ir = [tvr[v] + i for v in range(v_lo, v_hi)]
                    ico = [tvc[v] + j for v in range(v_lo, v_hi)]
                    for c in range(_NCG):
                        for vv, v in enumerate(range(v_lo, v_hi)):
                            out_v[bsel, kl, c, pl.ds(16 * v, 16)] = (
                                plsc.load_gather(
                                    in_cur, [cvecs[c], ir[vv], ico[vv]]))

            def piece_body(kp, cw):
                bsel = lax.rem(kp, 2)
                out_dma = pltpu.make_async_copy(
                    out_v.at[bsel],
                    out_hbm.at[pl.ds(kp * _KP, _KP), pl.ds(c0, _NCG),
                               pl.ds(p0, _PBLK)], sem_out)

                @pl.when(jnp.logical_or(slot > 0, kp >= 2))
                def _drain():
                    pltpu.make_async_copy(
                        out_v.at[bsel],
                        out_hbm.at[pl.ds(0, _KP), pl.ds(c0, _NCG),
                                   pl.ds(p0, _PBLK)], sem_out).wait()

                @pl.when(jnp.logical_not(crossing))
                def _normal():
                    gather_piece(kp, bsel, 0, 8)

                @pl.when(crossing)
                def _cross():
                    @pl.when(kp > 0)
                    def _restage1():
                        pltpu.sync_copy(
                            xpad_hbm.at[b0, pl.ds(c0, _NCG),
                                        pl.ds(216, 16), :],
                            in_v.at[par, :, pl.ds(0, 16), :])

                    gather_piece(kp, bsel, 0, 4)
                    pltpu.sync_copy(
                        xpad_hbm.at[b0 + 1, pl.ds(c0, _NCG),
                                    pl.ds(0, 16), :],
                        in_v.at[par, :, pl.ds(0, 16), :])
                    gather_piece(kp, bsel, 4, 8)

                out_dma.start()
                return cw

            lax.fori_loop(0, _NPIECE, piece_body, 0)

        return carry

    lax.fori_loop(0, _NSLOT, slot_body, 0)

    for _ in range(2):
        pltpu.make_async_copy(
            out_v.at[0],
            out_hbm.at[pl.ds(0, _KP), pl.ds(0, _NCG), pl.ds(0, _PBLK)],
            sem_out).wait()


def kernel(out_lr):
    xpad = jnp.pad(out_lr, ((0, 0), (0, 0), (_PAD, _HP - _H - _PAD),
                            (_PAD, _WPAD - _W - _PAD)))
    out_t = _extract_patches(xpad, jnp.asarray(_PTBL))
    patches = jnp.transpose(out_t.reshape(_K, _K, _C, _P), (3, 2, 0, 1))
    b_idx = jnp.repeat(jnp.arange(_B, dtype=jnp.int32), _NH * _NW)
    h_idx = jnp.tile(jnp.repeat(jnp.arange(_NH, dtype=jnp.int32), _NW), _B)
    w_idx = jnp.tile(jnp.arange(_NW, dtype=jnp.int32), _B * _NH)
    return (patches, (b_idx, h_idx, w_idx))
